# baseline (speedup 1.0000x reference)
import functools

import jax
import jax.numpy as jnp
import numpy as np
from jax import lax
from jax.experimental import pallas as pl
from jax.experimental.pallas import tpu as pltpu
from jax.experimental.pallas import tpu_sc as plsc

N = 10000
E = 320000
D = 128
H1 = 8
C1 = 8
H2 = 1
C2 = 16

NC = 2
NS = 16
L = 16
NW = NC * NS

NPAD = 10112
ROWS = NPAD // NS
EB = 48
NBUF = 4
NBLK = 212
EW = NBLK * EB
EPAD = NW * EW - E


def _mp_kernel(F, HH):
    CH = F // L
    C = F // HH
    shift = int(np.log2(C))

    def body(h_hbm, as_hbm, ad_hbm, src_hbm, dst_hbm, zf_hbm, zw_hbm,
             acc_out, den_out,
             src_b, dst_b, as_b, ad_b, w_b, h_b, acc_sh, den_sh,
             isem, gsem, ssem):
        c = lax.axis_index("c")
        s = lax.axis_index("s")
        wid = s * NC + c

        pltpu.sync_copy(zf_hbm, acc_sh.at[pl.ds(s * ROWS, ROWS)])
        pltpu.sync_copy(zw_hbm, den_sh.at[pl.ds(s * ROWS, ROWS)])
        plsc.subcore_barrier()

        def isu(b, p):
            row = wid * NBLK + b
            pltpu.async_copy(src_hbm.at[row], src_b.at[p], isem.at[p])
            pltpu.async_copy(dst_hbm.at[row], dst_b.at[p], isem.at[p])

        def iwait(b, p):
            row = wid * NBLK + b
            pltpu.make_async_copy(src_hbm.at[row], src_b.at[p],
                                  isem.at[p]).wait()
            pltpu.make_async_copy(dst_hbm.at[row], dst_b.at[p],
                                  isem.at[p]).wait()

        def gath(p):
            pltpu.async_copy(as_hbm.at[src_b.at[p]], as_b.at[p], gsem.at[p])
            pltpu.async_copy(ad_hbm.at[dst_b.at[p]], ad_b.at[p], gsem.at[p])
            pltpu.async_copy(h_hbm.at[src_b.at[p]], h_b.at[p], gsem.at[p])

        def gath_wait(p):
            pltpu.make_async_copy(as_hbm.at[src_b.at[p]], as_b.at[p],
                                  gsem.at[p]).wait()
            pltpu.make_async_copy(ad_hbm.at[dst_b.at[p]], ad_b.at[p],
                                  gsem.at[p]).wait()
            pltpu.make_async_copy(h_hbm.at[src_b.at[p]], h_b.at[p],
                                  gsem.at[p]).wait()

        def scat(p):
            pltpu.async_copy(h_b.at[p], acc_sh.at[dst_b.at[p]], ssem.at[p],
                             add=True)
            pltpu.async_copy(w_b.at[p], den_sh.at[dst_b.at[p]], ssem.at[p],
                             add=True)

        def scat_wait(p):
            pltpu.make_async_copy(h_b.at[p], acc_sh.at[dst_b.at[p]],
                                  ssem.at[p]).wait()
            pltpu.make_async_copy(w_b.at[p], den_sh.at[dst_b.at[p]],
                                  ssem.at[p]).wait()

        isu(0, 0)
        isu(1, 1)
        isu(2, 2)
        iwait(0, 0)
        gath(0)
        iwait(1, 1)
        gath(1)

        def outer(g, carry):
            for k in range(NBUF):
                b = g * NBUF + k
                gath_wait(k)

                @pl.when(b + 2 < NBLK)
                def _():
                    iwait(b + 2, (k + 2) % NBUF)
                    gath((k + 2) % NBUF)

                @pl.when(b >= 1)
                def _():
                    scat_wait((k + 3) % NBUF)

                @pl.when(b + 3 < NBLK)
                def _():
                    isu(b + 3, (k + 3) % NBUF)

                @plsc.parallel_loop(0, EB, 1, unroll=8)
                def edge(i):
                    e = as_b[k, i, :] + ad_b[k, i, :]
                    w = jnp.exp(jnp.where(e >= 0.0, e, 0.2 * e))
                    w_b[k, i, :] = w
                    lanes = lax.iota(jnp.int32, L)
                    for j in range(CH):
                        patj = lax.shift_right_logical(lanes + j * L, shift)
                        wb = lax.gather(
                            w, patj[:, None],
                            lax.GatherDimensionNumbers(
                                offset_dims=(), collapsed_slice_dims=(0,),
                                start_index_map=(0,)),
                            (1,), mode=lax.GatherScatterMode.PROMISE_IN_BOUNDS)
                        sl = pl.ds(j * L, L)
                        h_b[k, i, sl] = h_b[k, i, sl] * wb

                scat(k)
            return carry

        lax.fori_loop(0, NBLK // NBUF, outer, 0)
        scat_wait((NBLK - 1) % NBUF)
        plsc.subcore_barrier()
        sl = pl.ds(s * ROWS, ROWS)
        pltpu.sync_copy(acc_sh.at[sl], acc_out.at[c, sl])
        pltpu.sync_copy(den_sh.at[sl], den_out.at[c, sl])

    mesh = plsc.VectorSubcoreMesh(core_axis_name="c", subcore_axis_name="s",
                                  num_cores=NC, num_subcores=NS)
    return pl.kernel(
        body,
        out_type=(jax.ShapeDtypeStruct((NC, NPAD, F), jnp.float32),
                  jax.ShapeDtypeStruct((NC, NPAD, 16), jnp.float32)),
        mesh=mesh,
        compiler_params=pltpu.CompilerParams(use_tc_tiling_on_sc=False),
        scratch_types=(
            pltpu.VMEM((NBUF, EB), jnp.int32),
            pltpu.VMEM((NBUF, EB), jnp.int32),
            pltpu.VMEM((NBUF, EB, 16), jnp.float32),
            pltpu.VMEM((NBUF, EB, 16), jnp.float32),
            pltpu.VMEM((NBUF, EB, 16), jnp.float32),
            pltpu.VMEM((NBUF, EB, F), jnp.float32),
            pltpu.VMEM_SHARED((NPAD, F), jnp.float32),
            pltpu.VMEM_SHARED((NPAD, 16), jnp.float32),
            pltpu.SemaphoreType.DMA((NBUF,)),
            pltpu.SemaphoreType.DMA((NBUF,)),
            pltpu.SemaphoreType.DMA((NBUF,)),
        ),
    )


_GRID = 8
_BR = NPAD // _GRID


def _k1_body(x_ref, w_ref, h_ref, as_ref, ad_ref):
    y = jnp.dot(x_ref[:], w_ref[:], preferred_element_type=jnp.float32)
    h_ref[:] = y[:, :D]
    as_ref[:] = y[:, D:D + 16]
    ad_ref[:] = y[:, D + 16:D + 32]


def _k1(x_p, wc1):
    return pl.pallas_call(
        _k1_body,
        grid=(_GRID,),
        in_specs=[pl.BlockSpec((_BR, D), lambda i: (i, 0)),
                  pl.BlockSpec((D, D + 32), lambda i: (0, 0))],
        out_specs=[pl.BlockSpec((_BR, D), lambda i: (i, 0)),
                   pl.BlockSpec((_BR, 16), lambda i: (i, 0)),
                   pl.BlockSpec((_BR, 16), lambda i: (i, 0))],
        out_shape=[jax.ShapeDtypeStruct((NPAD, D), jnp.float32),
                   jax.ShapeDtypeStruct((NPAD, 16), jnp.float32),
                   jax.ShapeDtypeStruct((NPAD, 16), jnp.float32)],
    )(x_p, wc1)


def _elu(o):
    return jnp.where(o > 0.0, o, jnp.exp(o) - 1.0)


def _k2_body(accp, denp, r1, b1c, w2b, h2_ref, as_ref, ad_ref):
    acc = accp[0] + accp[1]
    den = denp[0] + denp[1] + 1e-16
    rep = jnp.dot(1.0 / den, r1[:], preferred_element_type=jnp.float32)
    o = acc * rep + b1c[:]
    e1 = _elu(o)
    x1 = e1[:, :H1 * C1] + e1[:, H1 * C1:]
    y2 = jnp.dot(x1, w2b[:], preferred_element_type=jnp.float32)
    h2_ref[:] = y2[:, :32]
    as_ref[:] = y2[:, 32:48]
    ad_ref[:] = y2[:, 48:64]


def _k2(acc1, den1, r1, b1c, w2b):
    return pl.pallas_call(
        _k2_body,
        grid=(_GRID,),
        in_specs=[pl.BlockSpec((NC, _BR, D), lambda i: (0, i, 0)),
                  pl.BlockSpec((NC, _BR, 16), lambda i: (0, i, 0)),
                  pl.BlockSpec((16, D), lambda i: (0, 0)),
                  pl.BlockSpec((1, D), lambda i: (0, 0)),
                  pl.BlockSpec((H1 * C1, 64), lambda i: (0, 0))],
        out_specs=[pl.BlockSpec((_BR, 32), lambda i: (i, 0)),
                   pl.BlockSpec((_BR, 16), lambda i: (i, 0)),
                   pl.BlockSpec((_BR, 16), lambda i: (i, 0))],
        out_shape=[jax.ShapeDtypeStruct((NPAD, 32), jnp.float32),
                   jax.ShapeDtypeStruct((NPAD, 16), jnp.float32),
                   jax.ShapeDtypeStruct((NPAD, 16), jnp.float32)],
    )(acc1, den1, r1, b1c, w2b)


def _k3_body(accp, denp, r2, b2c, out_ref):
    acc = accp[0] + accp[1]
    den = denp[0] + denp[1] + 1e-16
    rep = jnp.dot(1.0 / den, r2[:], preferred_element_type=jnp.float32)
    o = acc * rep + b2c[:]
    x2 = o[:, :C2] + o[:, C2:]
    m = jnp.max(x2, axis=1, keepdims=True)
    lse = jnp.log(jnp.sum(jnp.exp(x2 - m), axis=1, keepdims=True)) + m
    out_ref[:] = x2 - lse


def _k3(acc2, den2, r2, b2c):
    return pl.pallas_call(
        _k3_body,
        grid=(_GRID,),
        in_specs=[pl.BlockSpec((NC, _BR, 32), lambda i: (0, i, 0)),
                  pl.BlockSpec((NC, _BR, 16), lambda i: (0, i, 0)),
                  pl.BlockSpec((16, 32), lambda i: (0, 0)),
                  pl.BlockSpec((1, 32), lambda i: (0, 0))],
        out_specs=pl.BlockSpec((_BR, C2), lambda i: (i, 0)),
        out_shape=jax.ShapeDtypeStruct((NPAD, C2), jnp.float32),
    )(acc2, den2, r2, b2c)


def _fold(W, a_s, a_d, H, Cc):
    Wr = W.reshape(W.shape[0], H, Cc)
    ws = jnp.einsum("dhc,hc->dh", Wr, a_s)
    wd = jnp.einsum("dhc,hc->dh", Wr, a_d)
    return ws, wd


_R1 = np.zeros((16, D), np.float32)
for _h in range(16):
    _R1[_h, _h * 8:(_h + 1) * 8] = 1.0
_R2 = np.zeros((16, 32), np.float32)
_R2[0, :16] = 1.0
_R2[1, 16:] = 1.0


def kernel(x, edge_index, W1, as1, ad1, b1, W1r, as1r, ad1r, b1r,
           W2, as2, ad2, b2, W2r, as2r, ad2r, b2r):
    src = edge_index[0]
    dst = edge_index[1]
    pad_e = N + (jnp.arange(EPAD, dtype=jnp.int32) % (NPAD - N))
    srcp = jnp.concatenate([src, pad_e]).reshape(NW * NBLK, EB)
    dstp = jnp.concatenate([dst, pad_e]).reshape(NW * NBLK, EB)
    x_p = jnp.zeros((NPAD, D), jnp.float32).at[:N].set(x)

    ws1, wd1 = _fold(W1, as1, ad1, H1, C1)
    ws1r, wd1r = _fold(W1r, as1r, ad1r, H1, C1)
    wc1 = jnp.concatenate([W1, W1r, ws1, ws1r, wd1, wd1r], axis=1)

    ws2, wd2 = _fold(W2, as2, ad2, H2, C2)
    ws2r, wd2r = _fold(W2r, as2r, ad2r, H2, C2)
    z14 = jnp.zeros((H1 * C1, 14), jnp.float32)
    w2b = jnp.concatenate([W2, W2r, ws2, ws2r, z14, wd2, wd2r, z14], axis=1)

    b1c = jnp.concatenate([b1, b1r])[None, :]
    b2c = jnp.concatenate([b2, b2r])[None, :]

    zf1 = jnp.zeros((ROWS, D), jnp.float32)
    zf2 = jnp.zeros((ROWS, 32), jnp.float32)
    zw = jnp.zeros((ROWS, 16), jnp.float32)

    h1, as1t, ad1t = _k1(x_p, wc1)
    acc1, den1 = _mp_kernel(D, 16)(h1, as1t, ad1t, srcp, dstp, zf1, zw)
    h2, as2t, ad2t = _k2(acc1, den1, jnp.asarray(_R1), b1c, w2b)
    acc2, den2 = _mp_kernel(32, 2)(h2, as2t, ad2t, srcp, dstp, zf2, zw)
    out = _k3(acc2, den2, jnp.asarray(_R2), b2c)
    return out[:N]


# speedup vs baseline: 1.4455x; 1.4455x over previous
import functools

import jax
import jax.numpy as jnp
import numpy as np
from jax import lax
from jax.experimental import pallas as pl
from jax.experimental.pallas import tpu as pltpu
from jax.experimental.pallas import tpu_sc as plsc

N = 10000
E = 320000
D = 128
H1 = 8
C1 = 8
H2 = 1
C2 = 16

NC = 2
NS = 16
L = 16
NW = NC * NS

NPAD = 10112
ROWS = NPAD // NS
EB = 48
NBUF = 4
NBLK = 212
EW = NBLK * EB
EPAD = NW * EW - E


def _mp_kernel(F, HH):
    CH = F // L
    C = F // HH
    shift = int(np.log2(C))

    def body(h_hbm, as_hbm, ad_hbm, src_hbm, dst_hbm, zf_hbm, zw_hbm,
             acc_out, den_out,
             src_b, dst_b, as_b, ad_b, w_b, h_b, acc_sh, den_sh,
             isem, gsem, ssem):
        c = lax.axis_index("c")
        s = lax.axis_index("s")
        wid = s * NC + c

        pltpu.sync_copy(zf_hbm, acc_sh.at[pl.ds(s * ROWS, ROWS)])
        pltpu.sync_copy(zw_hbm, den_sh.at[pl.ds(s * ROWS, ROWS)])
        plsc.subcore_barrier()

        def isu(b, p):
            row = wid * NBLK + b
            pltpu.async_copy(src_hbm.at[row], src_b.at[p], isem.at[p])
            pltpu.async_copy(dst_hbm.at[row], dst_b.at[p], isem.at[p])

        def iwait(b, p):
            row = wid * NBLK + b
            pltpu.make_async_copy(src_hbm.at[row], src_b.at[p],
                                  isem.at[p]).wait()
            pltpu.make_async_copy(dst_hbm.at[row], dst_b.at[p],
                                  isem.at[p]).wait()

        def gath(p):
            pltpu.async_copy(as_hbm.at[src_b.at[p]], as_b.at[p], gsem.at[p])
            pltpu.async_copy(ad_hbm.at[dst_b.at[p]], ad_b.at[p], gsem.at[p])
            pltpu.async_copy(h_hbm.at[src_b.at[p]], h_b.at[p], gsem.at[p])

        def gath_wait(p):
            pltpu.make_async_copy(as_hbm.at[src_b.at[p]], as_b.at[p],
                                  gsem.at[p]).wait()
            pltpu.make_async_copy(ad_hbm.at[dst_b.at[p]], ad_b.at[p],
                                  gsem.at[p]).wait()
            pltpu.make_async_copy(h_hbm.at[src_b.at[p]], h_b.at[p],
                                  gsem.at[p]).wait()

        def scat(p):
            pltpu.async_copy(h_b.at[p], acc_sh.at[dst_b.at[p]], ssem.at[p],
                             add=True)
            pltpu.async_copy(w_b.at[p], den_sh.at[dst_b.at[p]], ssem.at[p],
                             add=True)

        def scat_wait(p):
            pltpu.make_async_copy(h_b.at[p], acc_sh.at[dst_b.at[p]],
                                  ssem.at[p]).wait()
            pltpu.make_async_copy(w_b.at[p], den_sh.at[dst_b.at[p]],
                                  ssem.at[p]).wait()

        isu(0, 0)
        isu(1, 1)
        isu(2, 2)
        iwait(0, 0)
        gath(0)
        iwait(1, 1)
        gath(1)

        def outer(g, carry):
            for k in range(NBUF):
                b = g * NBUF + k
                gath_wait(k)

                @pl.when(b + 2 < NBLK)
                def _():
                    iwait(b + 2, (k + 2) % NBUF)
                    gath((k + 2) % NBUF)

                @pl.when(b >= 1)
                def _():
                    scat_wait((k + 3) % NBUF)

                @pl.when(b + 3 < NBLK)
                def _():
                    isu(b + 3, (k + 3) % NBUF)

                @plsc.parallel_loop(0, EB, 1, unroll=4)
                def edge(i):
                    e = as_b[k, i, :] + ad_b[k, i, :]
                    w = jnp.exp(jnp.where(e >= 0.0, e, 0.2 * e))
                    w_b[k, i, :] = w
                    lanes = lax.iota(jnp.int32, L)
                    for j in range(CH):
                        patj = lax.shift_right_logical(lanes + j * L, shift)
                        wb = lax.gather(
                            w, patj[:, None],
                            lax.GatherDimensionNumbers(
                                offset_dims=(), collapsed_slice_dims=(0,),
                                start_index_map=(0,)),
                            (1,), mode=lax.GatherScatterMode.PROMISE_IN_BOUNDS)
                        sl = pl.ds(j * L, L)
                        h_b[k, i, sl] = h_b[k, i, sl] * wb

                scat(k)
            return carry

        lax.fori_loop(0, NBLK // NBUF, outer, 0)
        scat_wait((NBLK - 1) % NBUF)
        plsc.subcore_barrier()
        sl = pl.ds(s * ROWS, ROWS)
        pltpu.sync_copy(acc_sh.at[sl], acc_out.at[c, sl])
        pltpu.sync_copy(den_sh.at[sl], den_out.at[c, sl])

    mesh = plsc.VectorSubcoreMesh(core_axis_name="c", subcore_axis_name="s",
                                  num_cores=NC, num_subcores=NS)
    return pl.kernel(
        body,
        out_type=(jax.ShapeDtypeStruct((NC, NPAD, F), jnp.float32),
                  jax.ShapeDtypeStruct((NC, NPAD, 16), jnp.float32)),
        mesh=mesh,
        compiler_params=pltpu.CompilerParams(use_tc_tiling_on_sc=False),
        scratch_types=(
            pltpu.VMEM((NBUF, EB), jnp.int32),
            pltpu.VMEM((NBUF, EB), jnp.int32),
            pltpu.VMEM((NBUF, EB, 16), jnp.float32),
            pltpu.VMEM((NBUF, EB, 16), jnp.float32),
            pltpu.VMEM((NBUF, EB, 16), jnp.float32),
            pltpu.VMEM((NBUF, EB, F), jnp.float32),
            pltpu.VMEM_SHARED((NPAD, F), jnp.float32),
            pltpu.VMEM_SHARED((NPAD, 16), jnp.float32),
            pltpu.SemaphoreType.DMA((NBUF,)),
            pltpu.SemaphoreType.DMA((NBUF,)),
            pltpu.SemaphoreType.DMA((NBUF,)),
        ),
    )


_GRID = 8
_BR = NPAD // _GRID


def _k1_body(x_ref, w_ref, h_ref, as_ref, ad_ref):
    y = jnp.dot(x_ref[:], w_ref[:], preferred_element_type=jnp.float32)
    h_ref[:] = y[:, :D]
    as_ref[:] = y[:, D:D + 16]
    ad_ref[:] = y[:, D + 16:D + 32]


def _k1(x_p, wc1):
    return pl.pallas_call(
        _k1_body,
        grid=(_GRID,),
        in_specs=[pl.BlockSpec((_BR, D), lambda i: (i, 0)),
                  pl.BlockSpec((D, D + 32), lambda i: (0, 0))],
        out_specs=[pl.BlockSpec((_BR, D), lambda i: (i, 0)),
                   pl.BlockSpec((_BR, 16), lambda i: (i, 0)),
                   pl.BlockSpec((_BR, 16), lambda i: (i, 0))],
        out_shape=[jax.ShapeDtypeStruct((NPAD, D), jnp.float32),
                   jax.ShapeDtypeStruct((NPAD, 16), jnp.float32),
                   jax.ShapeDtypeStruct((NPAD, 16), jnp.float32)],
    )(x_p, wc1)


def _elu(o):
    return jnp.where(o > 0.0, o, jnp.exp(o) - 1.0)


def _k2_body(accp, denp, r1, b1c, w2b, h2_ref, as_ref, ad_ref):
    acc = accp[0] + accp[1]
    den = denp[0] + denp[1] + 1e-16
    rep = jnp.dot(1.0 / den, r1[:], preferred_element_type=jnp.float32)
    o = acc * rep + b1c[:]
    e1 = _elu(o)
    x1 = e1[:, :H1 * C1] + e1[:, H1 * C1:]
    y2 = jnp.dot(x1, w2b[:], preferred_element_type=jnp.float32)
    h2_ref[:] = y2[:, :32]
    as_ref[:] = y2[:, 32:48]
    ad_ref[:] = y2[:, 48:64]


def _k2(acc1, den1, r1, b1c, w2b):
    return pl.pallas_call(
        _k2_body,
        grid=(_GRID,),
        in_specs=[pl.BlockSpec((NC, _BR, D), lambda i: (0, i, 0)),
                  pl.BlockSpec((NC, _BR, 16), lambda i: (0, i, 0)),
                  pl.BlockSpec((16, D), lambda i: (0, 0)),
                  pl.BlockSpec((1, D), lambda i: (0, 0)),
                  pl.BlockSpec((H1 * C1, 64), lambda i: (0, 0))],
        out_specs=[pl.BlockSpec((_BR, 32), lambda i: (i, 0)),
                   pl.BlockSpec((_BR, 16), lambda i: (i, 0)),
                   pl.BlockSpec((_BR, 16), lambda i: (i, 0))],
        out_shape=[jax.ShapeDtypeStruct((NPAD, 32), jnp.float32),
                   jax.ShapeDtypeStruct((NPAD, 16), jnp.float32),
                   jax.ShapeDtypeStruct((NPAD, 16), jnp.float32)],
    )(acc1, den1, r1, b1c, w2b)


def _k3_body(accp, denp, r2, b2c, out_ref):
    acc = accp[0] + accp[1]
    den = denp[0] + denp[1] + 1e-16
    rep = jnp.dot(1.0 / den, r2[:], preferred_element_type=jnp.float32)
    o = acc * rep + b2c[:]
    x2 = o[:, :C2] + o[:, C2:]
    m = jnp.max(x2, axis=1, keepdims=True)
    lse = jnp.log(jnp.sum(jnp.exp(x2 - m), axis=1, keepdims=True)) + m
    out_ref[:] = x2 - lse


def _k3(acc2, den2, r2, b2c):
    return pl.pallas_call(
        _k3_body,
        grid=(_GRID,),
        in_specs=[pl.BlockSpec((NC, _BR, 32), lambda i: (0, i, 0)),
                  pl.BlockSpec((NC, _BR, 16), lambda i: (0, i, 0)),
                  pl.BlockSpec((16, 32), lambda i: (0, 0)),
                  pl.BlockSpec((1, 32), lambda i: (0, 0))],
        out_specs=pl.BlockSpec((_BR, C2), lambda i: (i, 0)),
        out_shape=jax.ShapeDtypeStruct((NPAD, C2), jnp.float32),
    )(acc2, den2, r2, b2c)


def _fold(W, a_s, a_d, H, Cc):
    Wr = W.reshape(W.shape[0], H, Cc)
    ws = jnp.einsum("dhc,hc->dh", Wr, a_s)
    wd = jnp.einsum("dhc,hc->dh", Wr, a_d)
    return ws, wd


_R1 = np.zeros((16, D), np.float32)
for _h in range(16):
    _R1[_h, _h * 8:(_h + 1) * 8] = 1.0
_R2 = np.zeros((16, 32), np.float32)
_R2[0, :16] = 1.0
_R2[1, 16:] = 1.0


def kernel(x, edge_index, W1, as1, ad1, b1, W1r, as1r, ad1r, b1r,
           W2, as2, ad2, b2, W2r, as2r, ad2r, b2r):
    src = edge_index[0]
    dst = edge_index[1]
    pad_e = N + (jnp.arange(EPAD, dtype=jnp.int32) % (NPAD - N))
    srcp = jnp.concatenate([src, pad_e]).reshape(NW * NBLK, EB)
    dstp = jnp.concatenate([dst, pad_e]).reshape(NW * NBLK, EB)
    x_p = jnp.zeros((NPAD, D), jnp.float32).at[:N].set(x)

    ws1, wd1 = _fold(W1, as1, ad1, H1, C1)
    ws1r, wd1r = _fold(W1r, as1r, ad1r, H1, C1)
    wc1 = jnp.concatenate([W1, W1r, ws1, ws1r, wd1, wd1r], axis=1)

    ws2, wd2 = _fold(W2, as2, ad2, H2, C2)
    ws2r, wd2r = _fold(W2r, as2r, ad2r, H2, C2)
    z14 = jnp.zeros((H1 * C1, 14), jnp.float32)
    w2b = jnp.concatenate([W2, W2r, ws2, ws2r, z14, wd2, wd2r, z14], axis=1)

    b1c = jnp.concatenate([b1, b1r])[None, :]
    b2c = jnp.concatenate([b2, b2r])[None, :]

    zf1 = jnp.zeros((ROWS, D), jnp.float32)
    zf2 = jnp.zeros((ROWS, 32), jnp.float32)
    zw = jnp.zeros((ROWS, 16), jnp.float32)

    h1, as1t, ad1t = _k1(x_p, wc1)
    acc1, den1 = _mp_kernel(D, 16)(h1, as1t, ad1t, srcp, dstp, zf1, zw)
    h2, as2t, ad2t = _k2(acc1, den1, jnp.asarray(_R1), b1c, w2b)
    acc2, den2 = _mp_kernel(32, 2)(h2, as2t, ad2t, srcp, dstp, zf2, zw)
    out = _k3(acc2, den2, jnp.asarray(_R2), b2c)
    return out[:N]


# speedup vs baseline: 1.4776x; 1.0222x over previous
import functools

import jax
import jax.numpy as jnp
import numpy as np
from jax import lax
from jax.experimental import pallas as pl
from jax.experimental.pallas import tpu as pltpu
from jax.experimental.pallas import tpu_sc as plsc

N = 10000
E = 320000
D = 128
H1 = 8
C1 = 8
H2 = 1
C2 = 16

NC = 2
NS = 16
L = 16
NW = NC * NS

NPAD = 10112
ROWS = NPAD // NS
EB = 56
NBUF = 4
NBLK = 180
EW = NBLK * EB
EPAD = NW * EW - E


def _mp_kernel(F, HH):
    FW = F + 16
    CH = F // L
    C = F // HH
    shift = int(np.log2(C))

    def body(hx_hbm, ad_hbm, src_hbm, dst_hbm, zf_hbm,
             acc_out,
             src_b, dst_b, ad_b, hx_b, acc_sh,
             isem, gsem, ssem):
        c = lax.axis_index("c")
        s = lax.axis_index("s")
        wid = s * NC + c

        pltpu.sync_copy(zf_hbm, acc_sh.at[pl.ds(s * ROWS, ROWS)])
        plsc.subcore_barrier()

        def isu(b, p):
            row = wid * NBLK + b
            pltpu.async_copy(src_hbm.at[row], src_b.at[p], isem.at[p])
            pltpu.async_copy(dst_hbm.at[row], dst_b.at[p], isem.at[p])

        def iwait(b, p):
            row = wid * NBLK + b
            pltpu.make_async_copy(src_hbm.at[row], src_b.at[p],
                                  isem.at[p]).wait()
            pltpu.make_async_copy(dst_hbm.at[row], dst_b.at[p],
                                  isem.at[p]).wait()

        def gath(p):
            pltpu.async_copy(ad_hbm.at[dst_b.at[p]], ad_b.at[p], gsem.at[p])
            pltpu.async_copy(hx_hbm.at[src_b.at[p]], hx_b.at[p], gsem.at[p])

        def gath_wait(p):
            pltpu.make_async_copy(ad_hbm.at[dst_b.at[p]], ad_b.at[p],
                                  gsem.at[p]).wait()
            pltpu.make_async_copy(hx_hbm.at[src_b.at[p]], hx_b.at[p],
                                  gsem.at[p]).wait()

        def scat(p):
            pltpu.async_copy(hx_b.at[p], acc_sh.at[dst_b.at[p]], ssem.at[p],
                             add=True)

        def scat_wait(p):
            pltpu.make_async_copy(hx_b.at[p], acc_sh.at[dst_b.at[p]],
                                  ssem.at[p]).wait()

        isu(0, 0)
        isu(1, 1)
        isu(2, 2)
        iwait(0, 0)
        gath(0)
        iwait(1, 1)
        gath(1)

        def outer(g, carry):
            for k in range(NBUF):
                b = g * NBUF + k
                gath_wait(k)

                @pl.when(b + 2 < NBLK)
                def _():
                    iwait(b + 2, (k + 2) % NBUF)
                    gath((k + 2) % NBUF)

                @pl.when(b >= 1)
                def _():
                    scat_wait((k + 3) % NBUF)

                @pl.when(b + 3 < NBLK)
                def _():
                    isu(b + 3, (k + 3) % NBUF)

                @plsc.parallel_loop(0, EB, 1, unroll=4)
                def edge(i):
                    e = hx_b[k, i, pl.ds(F, L)] + ad_b[k, i, :]
                    w = jnp.exp(jnp.where(e >= 0.0, e, 0.2 * e))
                    hx_b[k, i, pl.ds(F, L)] = w
                    lanes = lax.iota(jnp.int32, L)
                    for j in range(CH):
                        patj = lax.shift_right_logical(lanes + j * L, shift)
                        wb = lax.gather(
                            w, patj[:, None],
                            lax.GatherDimensionNumbers(
                                offset_dims=(), collapsed_slice_dims=(0,),
                                start_index_map=(0,)),
                            (1,), mode=lax.GatherScatterMode.PROMISE_IN_BOUNDS)
                        sl = pl.ds(j * L, L)
                        hx_b[k, i, sl] = hx_b[k, i, sl] * wb

                scat(k)
            return carry

        lax.fori_loop(0, NBLK // NBUF, outer, 0)
        scat_wait((NBLK - 1) % NBUF)
        plsc.subcore_barrier()
        sl = pl.ds(s * ROWS, ROWS)
        pltpu.sync_copy(acc_sh.at[sl], acc_out.at[c, sl])

    mesh = plsc.VectorSubcoreMesh(core_axis_name="c", subcore_axis_name="s",
                                  num_cores=NC, num_subcores=NS)
    return pl.kernel(
        body,
        out_type=jax.ShapeDtypeStruct((NC, NPAD, FW), jnp.float32),
        mesh=mesh,
        compiler_params=pltpu.CompilerParams(use_tc_tiling_on_sc=False),
        scratch_types=(
            pltpu.VMEM((NBUF, EB), jnp.int32),
            pltpu.VMEM((NBUF, EB), jnp.int32),
            pltpu.VMEM((NBUF, EB, 16), jnp.float32),
            pltpu.VMEM((NBUF, EB, FW), jnp.float32),
            pltpu.VMEM_SHARED((NPAD, FW), jnp.float32),
            pltpu.SemaphoreType.DMA((NBUF,)),
            pltpu.SemaphoreType.DMA((NBUF,)),
            pltpu.SemaphoreType.DMA((NBUF,)),
        ),
    )


_GRID = 8
_BR = NPAD // _GRID


def _k1_body(x_ref, w_ref, hx_ref, ad_ref):
    y = jnp.dot(x_ref[:], w_ref[:], preferred_element_type=jnp.float32)
    hx_ref[:] = y[:, :D + 16]
    ad_ref[:] = y[:, D + 16:D + 32]


def _k1(x_p, wc1):
    return pl.pallas_call(
        _k1_body,
        grid=(_GRID,),
        in_specs=[pl.BlockSpec((_BR, D), lambda i: (i, 0)),
                  pl.BlockSpec((D, D + 32), lambda i: (0, 0))],
        out_specs=[pl.BlockSpec((_BR, D + 16), lambda i: (i, 0)),
                   pl.BlockSpec((_BR, 16), lambda i: (i, 0))],
        out_shape=[jax.ShapeDtypeStruct((NPAD, D + 16), jnp.float32),
                   jax.ShapeDtypeStruct((NPAD, 16), jnp.float32)],
    )(x_p, wc1)


def _elu(o):
    return jnp.where(o > 0.0, o, jnp.exp(o) - 1.0)


def _k2_body(accp, r1, b1c, w2b, hx2_ref, ad_ref):
    acc = accp[0] + accp[1]
    den = acc[:, D:D + 16] + 1e-16
    rep = jnp.dot(1.0 / den, r1[:], preferred_element_type=jnp.float32)
    o = acc[:, :D] * rep + b1c[:]
    e1 = _elu(o)
    x1 = e1[:, :H1 * C1] + e1[:, H1 * C1:]
    y2 = jnp.dot(x1, w2b[:], preferred_element_type=jnp.float32)
    hx2_ref[:] = y2[:, :48]
    ad_ref[:] = y2[:, 48:64]


def _k2(acc1, r1, b1c, w2b):
    return pl.pallas_call(
        _k2_body,
        grid=(_GRID,),
        in_specs=[pl.BlockSpec((NC, _BR, D + 16), lambda i: (0, i, 0)),
                  pl.BlockSpec((16, D), lambda i: (0, 0)),
                  pl.BlockSpec((1, D), lambda i: (0, 0)),
                  pl.BlockSpec((H1 * C1, 64), lambda i: (0, 0))],
        out_specs=[pl.BlockSpec((_BR, 48), lambda i: (i, 0)),
                   pl.BlockSpec((_BR, 16), lambda i: (i, 0))],
        out_shape=[jax.ShapeDtypeStruct((NPAD, 48), jnp.float32),
                   jax.ShapeDtypeStruct((NPAD, 16), jnp.float32)],
    )(acc1, r1, b1c, w2b)


def _k3_body(accp, r2, b2c, out_ref):
    acc = accp[0] + accp[1]
    den = acc[:, 32:48] + 1e-16
    rep = jnp.dot(1.0 / den, r2[:], preferred_element_type=jnp.float32)
    o = acc[:, :32] * rep + b2c[:]
    x2 = o[:, :C2] + o[:, C2:]
    m = jnp.max(x2, axis=1, keepdims=True)
    lse = jnp.log(jnp.sum(jnp.exp(x2 - m), axis=1, keepdims=True)) + m
    out_ref[:] = x2 - lse


def _k3(acc2, r2, b2c):
    return pl.pallas_call(
        _k3_body,
        grid=(_GRID,),
        in_specs=[pl.BlockSpec((NC, _BR, 48), lambda i: (0, i, 0)),
                  pl.BlockSpec((16, 32), lambda i: (0, 0)),
                  pl.BlockSpec((1, 32), lambda i: (0, 0))],
        out_specs=pl.BlockSpec((_BR, C2), lambda i: (i, 0)),
        out_shape=jax.ShapeDtypeStruct((NPAD, C2), jnp.float32),
    )(acc2, r2, b2c)


def _fold(W, a_s, a_d, H, Cc):
    Wr = W.reshape(W.shape[0], H, Cc)
    ws = jnp.einsum("dhc,hc->dh", Wr, a_s)
    wd = jnp.einsum("dhc,hc->dh", Wr, a_d)
    return ws, wd


_R1 = np.zeros((16, D), np.float32)
for _h in range(16):
    _R1[_h, _h * 8:(_h + 1) * 8] = 1.0
_R2 = np.zeros((16, 32), np.float32)
_R2[0, :16] = 1.0
_R2[1, 16:] = 1.0


def kernel(x, edge_index, W1, as1, ad1, b1, W1r, as1r, ad1r, b1r,
           W2, as2, ad2, b2, W2r, as2r, ad2r, b2r):
    src = edge_index[0]
    dst = edge_index[1]
    pad_e = N + (jnp.arange(EPAD, dtype=jnp.int32) % (NPAD - N))
    srcp = jnp.concatenate([src, pad_e]).reshape(NW * NBLK, EB)
    dstp = jnp.concatenate([dst, pad_e]).reshape(NW * NBLK, EB)
    x_p = jnp.zeros((NPAD, D), jnp.float32).at[:N].set(x)

    ws1, wd1 = _fold(W1, as1, ad1, H1, C1)
    ws1r, wd1r = _fold(W1r, as1r, ad1r, H1, C1)
    wc1 = jnp.concatenate([W1, W1r, ws1, ws1r, wd1, wd1r], axis=1)

    ws2, wd2 = _fold(W2, as2, ad2, H2, C2)
    ws2r, wd2r = _fold(W2r, as2r, ad2r, H2, C2)
    z14 = jnp.zeros((H1 * C1, 14), jnp.float32)
    w2b = jnp.concatenate([W2, W2r, ws2, ws2r, z14, wd2, wd2r, z14], axis=1)

    b1c = jnp.concatenate([b1, b1r])[None, :]
    b2c = jnp.concatenate([b2, b2r])[None, :]

    zf1 = jnp.zeros((ROWS, D + 16), jnp.float32)
    zf2 = jnp.zeros((ROWS, 48), jnp.float32)

    hx1, ad1t = _k1(x_p, wc1)
    acc1 = _mp_kernel(D, 16)(hx1, ad1t, srcp, dstp, zf1)
    hx2, ad2t = _k2(acc1, jnp.asarray(_R1), b1c, w2b)
    acc2 = _mp_kernel(32, 2)(hx2, ad2t, srcp, dstp, zf2)
    out = _k3(acc2, jnp.asarray(_R2), b2c)
    return out[:N]


# speedup vs baseline: 1.6110x; 1.0903x over previous
import functools

import jax
import jax.numpy as jnp
import numpy as np
from jax import lax
from jax.experimental import pallas as pl
from jax.experimental.pallas import tpu as pltpu
from jax.experimental.pallas import tpu_sc as plsc

N = 10000
E = 320000
D = 128
H1 = 8
C1 = 8
H2 = 1
C2 = 16

NC = 2
NS = 16
L = 16
NW = NC * NS

NPAD = 10112
ROWS = NPAD // NS
NBUF = 4
EB1, NBLK1 = 56, 180
EB2, NBLK2 = 128, 80


def _mp_kernel(F, HH, EB, NBLK):
    FW = F + 16
    CH = F // L
    C = F // HH
    shift = int(np.log2(C))

    def body(hx_hbm, ad_hbm, src_hbm, dst_hbm, zf_hbm,
             acc_out,
             src_b, dst_b, ad_b, hx_b, acc_sh,
             isem, gsem, ssem):
        c = lax.axis_index("c")
        s = lax.axis_index("s")
        wid = s * NC + c

        pltpu.sync_copy(zf_hbm, acc_sh.at[pl.ds(s * ROWS, ROWS)])
        plsc.subcore_barrier()

        def isu(b, p):
            row = wid * NBLK + b
            pltpu.async_copy(src_hbm.at[row], src_b.at[p], isem.at[p])
            pltpu.async_copy(dst_hbm.at[row], dst_b.at[p], isem.at[p])

        def iwait(b, p):
            row = wid * NBLK + b
            pltpu.make_async_copy(src_hbm.at[row], src_b.at[p],
                                  isem.at[p]).wait()
            pltpu.make_async_copy(dst_hbm.at[row], dst_b.at[p],
                                  isem.at[p]).wait()

        def gath(p):
            pltpu.async_copy(ad_hbm.at[dst_b.at[p]], ad_b.at[p], gsem.at[p])
            pltpu.async_copy(hx_hbm.at[src_b.at[p]], hx_b.at[p], gsem.at[p])

        def gath_wait(p):
            pltpu.make_async_copy(ad_hbm.at[dst_b.at[p]], ad_b.at[p],
                                  gsem.at[p]).wait()
            pltpu.make_async_copy(hx_hbm.at[src_b.at[p]], hx_b.at[p],
                                  gsem.at[p]).wait()

        def scat(p):
            pltpu.async_copy(hx_b.at[p], acc_sh.at[dst_b.at[p]], ssem.at[p],
                             add=True)

        def scat_wait(p):
            pltpu.make_async_copy(hx_b.at[p], acc_sh.at[dst_b.at[p]],
                                  ssem.at[p]).wait()

        isu(0, 0)
        isu(1, 1)
        isu(2, 2)
        iwait(0, 0)
        gath(0)
        iwait(1, 1)
        gath(1)

        def outer(g, carry):
            for k in range(NBUF):
                b = g * NBUF + k
                gath_wait(k)

                @pl.when(b + 2 < NBLK)
                def _():
                    iwait(b + 2, (k + 2) % NBUF)
                    gath((k + 2) % NBUF)

                @pl.when(b >= 1)
                def _():
                    scat_wait((k + 3) % NBUF)

                @pl.when(b + 3 < NBLK)
                def _():
                    isu(b + 3, (k + 3) % NBUF)

                @plsc.parallel_loop(0, EB, 1, unroll=4)
                def edge(i):
                    e = hx_b[k, i, pl.ds(F, L)] + ad_b[k, i, :]
                    w = jnp.exp(jnp.where(e >= 0.0, e, 0.2 * e))
                    hx_b[k, i, pl.ds(F, L)] = w
                    lanes = lax.iota(jnp.int32, L)
                    for j in range(CH):
                        patj = lax.shift_right_logical(lanes + j * L, shift)
                        wb = lax.gather(
                            w, patj[:, None],
                            lax.GatherDimensionNumbers(
                                offset_dims=(), collapsed_slice_dims=(0,),
                                start_index_map=(0,)),
                            (1,), mode=lax.GatherScatterMode.PROMISE_IN_BOUNDS)
                        sl = pl.ds(j * L, L)
                        hx_b[k, i, sl] = hx_b[k, i, sl] * wb

                scat(k)
            return carry

        lax.fori_loop(0, NBLK // NBUF, outer, 0)
        scat_wait((NBLK - 1) % NBUF)
        plsc.subcore_barrier()
        sl = pl.ds(s * ROWS, ROWS)
        pltpu.sync_copy(acc_sh.at[sl], acc_out.at[c, sl])

    mesh = plsc.VectorSubcoreMesh(core_axis_name="c", subcore_axis_name="s",
                                  num_cores=NC, num_subcores=NS)
    return pl.kernel(
        body,
        out_type=jax.ShapeDtypeStruct((NC, NPAD, FW), jnp.float32),
        mesh=mesh,
        compiler_params=pltpu.CompilerParams(use_tc_tiling_on_sc=False),
        scratch_types=(
            pltpu.VMEM((NBUF, EB), jnp.int32),
            pltpu.VMEM((NBUF, EB), jnp.int32),
            pltpu.VMEM((NBUF, EB, 16), jnp.float32),
            pltpu.VMEM((NBUF, EB, FW), jnp.float32),
            pltpu.VMEM_SHARED((NPAD, FW), jnp.float32),
            pltpu.SemaphoreType.DMA((NBUF,)),
            pltpu.SemaphoreType.DMA((NBUF,)),
            pltpu.SemaphoreType.DMA((NBUF,)),
        ),
    )


_GRID = 8
_BR = NPAD // _GRID


def _k1_body(x_ref, w_ref, hx_ref, ad_ref):
    y = jnp.dot(x_ref[:], w_ref[:], preferred_element_type=jnp.float32)
    hx_ref[:] = y[:, :D + 16]
    ad_ref[:] = y[:, D + 16:D + 32]


def _k1(x_p, wc1):
    return pl.pallas_call(
        _k1_body,
        grid=(_GRID,),
        in_specs=[pl.BlockSpec((_BR, D), lambda i: (i, 0)),
                  pl.BlockSpec((D, D + 32), lambda i: (0, 0))],
        out_specs=[pl.BlockSpec((_BR, D + 16), lambda i: (i, 0)),
                   pl.BlockSpec((_BR, 16), lambda i: (i, 0))],
        out_shape=[jax.ShapeDtypeStruct((NPAD, D + 16), jnp.float32),
                   jax.ShapeDtypeStruct((NPAD, 16), jnp.float32)],
    )(x_p, wc1)


def _elu(o):
    return jnp.where(o > 0.0, o, jnp.exp(o) - 1.0)


def _k2_body(accp, r1, b1c, w2b, hx2_ref, ad_ref):
    acc = accp[0] + accp[1]
    den = acc[:, D:D + 16] + 1e-16
    rep = jnp.dot(1.0 / den, r1[:], preferred_element_type=jnp.float32)
    o = acc[:, :D] * rep + b1c[:]
    e1 = _elu(o)
    x1 = e1[:, :H1 * C1] + e1[:, H1 * C1:]
    y2 = jnp.dot(x1, w2b[:], preferred_element_type=jnp.float32)
    hx2_ref[:] = y2[:, :48]
    ad_ref[:] = y2[:, 48:64]


def _k2(acc1, r1, b1c, w2b):
    return pl.pallas_call(
        _k2_body,
        grid=(_GRID,),
        in_specs=[pl.BlockSpec((NC, _BR, D + 16), lambda i: (0, i, 0)),
                  pl.BlockSpec((16, D), lambda i: (0, 0)),
                  pl.BlockSpec((1, D), lambda i: (0, 0)),
                  pl.BlockSpec((H1 * C1, 64), lambda i: (0, 0))],
        out_specs=[pl.BlockSpec((_BR, 48), lambda i: (i, 0)),
                   pl.BlockSpec((_BR, 16), lambda i: (i, 0))],
        out_shape=[jax.ShapeDtypeStruct((NPAD, 48), jnp.float32),
                   jax.ShapeDtypeStruct((NPAD, 16), jnp.float32)],
    )(acc1, r1, b1c, w2b)


def _k3_body(accp, r2, b2c, out_ref):
    acc = accp[0] + accp[1]
    den = acc[:, 32:48] + 1e-16
    rep = jnp.dot(1.0 / den, r2[:], preferred_element_type=jnp.float32)
    o = acc[:, :32] * rep + b2c[:]
    x2 = o[:, :C2] + o[:, C2:]
    m = jnp.max(x2, axis=1, keepdims=True)
    lse = jnp.log(jnp.sum(jnp.exp(x2 - m), axis=1, keepdims=True)) + m
    out_ref[:] = x2 - lse


def _k3(acc2, r2, b2c):
    return pl.pallas_call(
        _k3_body,
        grid=(_GRID,),
        in_specs=[pl.BlockSpec((NC, _BR, 48), lambda i: (0, i, 0)),
                  pl.BlockSpec((16, 32), lambda i: (0, 0)),
                  pl.BlockSpec((1, 32), lambda i: (0, 0))],
        out_specs=pl.BlockSpec((_BR, C2), lambda i: (i, 0)),
        out_shape=jax.ShapeDtypeStruct((NPAD, C2), jnp.float32),
    )(acc2, r2, b2c)


def _fold(W, a_s, a_d, H, Cc):
    Wr = W.reshape(W.shape[0], H, Cc)
    ws = jnp.einsum("dhc,hc->dh", Wr, a_s)
    wd = jnp.einsum("dhc,hc->dh", Wr, a_d)
    return ws, wd


_R1 = np.zeros((16, D), np.float32)
for _h in range(16):
    _R1[_h, _h * 8:(_h + 1) * 8] = 1.0
_R2 = np.zeros((16, 32), np.float32)
_R2[0, :16] = 1.0
_R2[1, 16:] = 1.0


def kernel(x, edge_index, W1, as1, ad1, b1, W1r, as1r, ad1r, b1r,
           W2, as2, ad2, b2, W2r, as2r, ad2r, b2r):
    src = edge_index[0]
    dst = edge_index[1]
    def pad_edges(v, eb, nblk):
        epad = NW * nblk * eb - E
        pad_e = N + (jnp.arange(epad, dtype=jnp.int32) % (NPAD - N))
        return jnp.concatenate([v, pad_e]).reshape(NW * nblk, eb)

    src1 = pad_edges(src, EB1, NBLK1)
    dst1 = pad_edges(dst, EB1, NBLK1)
    src2 = pad_edges(src, EB2, NBLK2)
    dst2 = pad_edges(dst, EB2, NBLK2)
    x_p = jnp.zeros((NPAD, D), jnp.float32).at[:N].set(x)

    ws1, wd1 = _fold(W1, as1, ad1, H1, C1)
    ws1r, wd1r = _fold(W1r, as1r, ad1r, H1, C1)
    wc1 = jnp.concatenate([W1, W1r, ws1, ws1r, wd1, wd1r], axis=1)

    ws2, wd2 = _fold(W2, as2, ad2, H2, C2)
    ws2r, wd2r = _fold(W2r, as2r, ad2r, H2, C2)
    z14 = jnp.zeros((H1 * C1, 14), jnp.float32)
    w2b = jnp.concatenate([W2, W2r, ws2, ws2r, z14, wd2, wd2r, z14], axis=1)

    b1c = jnp.concatenate([b1, b1r])[None, :]
    b2c = jnp.concatenate([b2, b2r])[None, :]

    zf1 = jnp.zeros((ROWS, D + 16), jnp.float32)
    zf2 = jnp.zeros((ROWS, 48), jnp.float32)

    hx1, ad1t = _k1(x_p, wc1)
    acc1 = _mp_kernel(D, 16, EB1, NBLK1)(hx1, ad1t, src1, dst1, zf1)
    hx2, ad2t = _k2(acc1, jnp.asarray(_R1), b1c, w2b)
    acc2 = _mp_kernel(32, 2, EB2, NBLK2)(hx2, ad2t, src2, dst2, zf2)
    out = _k3(acc2, jnp.asarray(_R2), b2c)
    return out[:N]
